# baseline (speedup 1.0000x reference)
import jax
import jax.numpy as jnp
from jax.experimental import pallas as pl
from jax.experimental.pallas import tpu as pltpu

_N_EXP = 16
_TOP_K = 2
_M = 128
_F = 128


def _router_kernel(xf_ref, gw_ref, gb_ref, comb_ref, rank_ref, rankt_ref,
                   cnt_ref):
    scores = jnp.dot(xf_ref[...], gw_ref[...],
                     preferred_element_type=jnp.float32) + gb_ref[...]
    T = scores.shape[0]
    col = jax.lax.broadcasted_iota(jnp.int32, scores.shape, 1)
    m1 = jnp.max(scores, axis=1, keepdims=True)
    idx1 = jnp.min(jnp.where(scores == m1, col, _N_EXP), axis=1, keepdims=True)
    masked = jnp.where(col == idx1, -jnp.inf, scores)
    m2 = jnp.max(masked, axis=1, keepdims=True)
    idx2 = jnp.min(jnp.where(masked == m2, col, _N_EXP), axis=1, keepdims=True)
    oh1 = (col == idx1).astype(jnp.float32)
    oh2 = (col == idx2).astype(jnp.float32)
    mask = oh1 + oh2
    comb_ref[...] = oh1 * m1 + oh2 * m2

    CH = 256 if T % 256 == 0 else T
    r_i = jax.lax.broadcasted_iota(jnp.int32, (CH, CH), 0)
    c_i = jax.lax.broadcasted_iota(jnp.int32, (CH, CH), 1)
    ltri = (c_i < r_i).astype(jnp.float32)
    chunks = []
    prefix = jnp.zeros((1, _N_EXP), jnp.float32)
    for ci in range(T // CH):
        sub = mask[ci * CH:(ci + 1) * CH]
        cum = jnp.dot(ltri, sub, preferred_element_type=jnp.float32) + prefix
        chunks.append(cum)
        prefix = prefix + jnp.sum(sub, axis=0, keepdims=True)
    rank = jnp.concatenate(chunks, axis=0)
    rankm = jnp.where(mask > 0, rank, -1.0)
    rank_ref[...] = rankm
    rankt_ref[...] = rankm.T
    cnt_ref[...] = prefix.astype(jnp.int32)


def _moe_kernel(cnt_ref, comb_ref, rank_ref, rankt_ref, xf_ref,
                wup_ref, bup_ref, wg_ref, bg_ref, wdn_ref, bdn_ref,
                out_ref, xblk_ref, acc_ref):
    e = pl.program_id(0)
    f = pl.program_id(1)
    nf = pl.num_programs(1)
    ce = cnt_ref[0, e]
    nb = (ce + _M - 1) // _M
    T = xf_ref.shape[0]

    @pl.when((e == 0) & (f == 0))
    def _init():
        out_ref[...] = jnp.zeros_like(out_ref)

    @pl.when(f == 0)
    def _gather():
        row = jax.lax.broadcasted_iota(jnp.int32, (1, T), 0)

        def erow(rt):
            return jnp.sum(
                jnp.where(jax.lax.broadcasted_iota(
                    jnp.int32, rt.shape, 0) == e, rt, 0.0),
                axis=0, keepdims=True)

        rank_row = erow(rankt_ref[...])
        riota = jax.lax.broadcasted_iota(
            jnp.int32, (_M, T), 0).astype(jnp.float32)

        def gath(rb, c):
            oh = (rank_row == riota + (rb * _M).astype(jnp.float32)
                  ).astype(jnp.float32)
            xg = jnp.dot(oh, xf_ref[...], preferred_element_type=jnp.float32)
            xblk_ref[pl.ds(rb * _M, _M), :] = xg
            return c

        jax.lax.fori_loop(0, nb, gath, 0)

    bup = bup_ref[0, 0]
    bg = bg_ref[0, 0]
    bdn = bdn_ref[0]
    isf0 = f == 0

    def comp(rb, c):
        xb = xblk_ref[pl.ds(rb * _M, _M), :]
        u = jnp.dot(xb, wup_ref[0], preferred_element_type=jnp.float32) + bup
        g = jnp.dot(xb, wg_ref[0], preferred_element_type=jnp.float32) + bg
        h = u * (g * jax.lax.logistic(g))
        p = jnp.dot(h, wdn_ref[0], preferred_element_type=jnp.float32)
        prev = acc_ref[pl.ds(rb * _M, _M), :]
        acc_ref[pl.ds(rb * _M, _M), :] = p + jnp.where(
            isf0, jnp.broadcast_to(bdn, p.shape), prev)
        return c

    jax.lax.fori_loop(0, nb, comp, 0)

    @pl.when(f == nf - 1)
    def _scatter():
        col16 = jax.lax.broadcasted_iota(jnp.int32, (T, _N_EXP), 1)
        rank_col = jnp.sum(
            jnp.where(col16 == e, rank_ref[...], 0.0), axis=1, keepdims=True)
        comb_col = jnp.sum(
            jnp.where(col16 == e, comb_ref[...], 0.0), axis=1, keepdims=True)
        siota = jax.lax.broadcasted_iota(
            jnp.int32, (T, _M), 1).astype(jnp.float32)

        def scat(rb, c):
            oh = (rank_col == siota + (rb * _M).astype(jnp.float32)
                  ).astype(jnp.float32)
            out_ref[...] += jnp.dot(oh * comb_col,
                                    acc_ref[pl.ds(rb * _M, _M), :],
                                    preferred_element_type=jnp.float32)
            return c

        jax.lax.fori_loop(0, nb, scat, 0)


def kernel(x, gate_W, gate_b, W_up, b_up, W_g, b_g, W_down, b_down):
    B, T, C = x.shape
    D_FF = W_up.shape[2]
    NF = D_FF // _F
    TP = ((T + _M - 1) // _M) * _M
    xf = x.reshape(T, C)

    comb, rankm, rankmt, cnt = pl.pallas_call(
        _router_kernel,
        out_shape=(jax.ShapeDtypeStruct((T, _N_EXP), jnp.float32),
                   jax.ShapeDtypeStruct((T, _N_EXP), jnp.float32),
                   jax.ShapeDtypeStruct((_N_EXP, T), jnp.float32),
                   jax.ShapeDtypeStruct((1, _N_EXP), jnp.int32)),
    )(xf, gate_W, gate_b.reshape(1, _N_EXP))

    smem = pl.BlockSpec(memory_space=pltpu.SMEM)
    out = pl.pallas_call(
        _moe_kernel,
        grid=(_N_EXP, NF),
        in_specs=[
            smem,
            pl.BlockSpec((T, _N_EXP), lambda e, f: (0, 0)),
            pl.BlockSpec((T, _N_EXP), lambda e, f: (0, 0)),
            pl.BlockSpec((_N_EXP, T), lambda e, f: (0, 0)),
            pl.BlockSpec((T, C), lambda e, f: (0, 0)),
            pl.BlockSpec((1, C, _F), lambda e, f: (e, 0, f)),
            pl.BlockSpec((1, 1, 1, _F), lambda e, f: (e, f, 0, 0)),
            pl.BlockSpec((1, C, _F), lambda e, f: (e, 0, f)),
            pl.BlockSpec((1, 1, 1, _F), lambda e, f: (e, f, 0, 0)),
            pl.BlockSpec((1, _F, C), lambda e, f: (e, f, 0)),
            pl.BlockSpec((1, 1, C), lambda e, f: (e, 0, 0)),
        ],
        out_specs=pl.BlockSpec((T, C), lambda e, f: (0, 0)),
        out_shape=jax.ShapeDtypeStruct((T, C), jnp.float32),
        scratch_shapes=[pltpu.VMEM((TP, C), jnp.float32),
                        pltpu.VMEM((TP, C), jnp.float32)],
    )(cnt, comb, rankm, rankmt, xf, W_up, b_up.reshape(_N_EXP, NF, 1, _F),
      W_g, b_g.reshape(_N_EXP, NF, 1, _F), W_down,
      b_down.reshape(_N_EXP, 1, C))

    return out.reshape(B, T, C)


# speedup vs baseline: 1.5389x; 1.5389x over previous
import functools

import jax
import jax.numpy as jnp
from jax.experimental import pallas as pl
from jax.experimental.pallas import tpu as pltpu

_N_EXP = 16
_TOP_K = 2
_M = 128
_F = 512


def _router_kernel(xf_ref, gw_ref, gb_ref, comb_ref, rank_ref, rankt_ref,
                   cnt_ref):
    scores = jnp.dot(xf_ref[...], gw_ref[...],
                     preferred_element_type=jnp.float32) + gb_ref[...]
    T = scores.shape[0]
    col = jax.lax.broadcasted_iota(jnp.int32, scores.shape, 1)
    m1 = jnp.max(scores, axis=1, keepdims=True)
    idx1 = jnp.min(jnp.where(scores == m1, col, _N_EXP), axis=1, keepdims=True)
    masked = jnp.where(col == idx1, -jnp.inf, scores)
    m2 = jnp.max(masked, axis=1, keepdims=True)
    idx2 = jnp.min(jnp.where(masked == m2, col, _N_EXP), axis=1, keepdims=True)
    oh1 = (col == idx1).astype(jnp.float32)
    oh2 = (col == idx2).astype(jnp.float32)
    mask = oh1 + oh2
    comb_ref[...] = oh1 * m1 + oh2 * m2

    CH = 256 if T % 256 == 0 else T
    r_i = jax.lax.broadcasted_iota(jnp.int32, (CH, CH), 0)
    c_i = jax.lax.broadcasted_iota(jnp.int32, (CH, CH), 1)
    ltri = (c_i < r_i).astype(jnp.float32)
    chunks = []
    prefix = jnp.zeros((1, _N_EXP), jnp.float32)
    for ci in range(T // CH):
        sub = mask[ci * CH:(ci + 1) * CH]
        cum = jnp.dot(ltri, sub, preferred_element_type=jnp.float32) + prefix
        chunks.append(cum)
        prefix = prefix + jnp.sum(sub, axis=0, keepdims=True)
    rank = jnp.concatenate(chunks, axis=0)
    rankm = jnp.where(mask > 0, rank, -1.0)
    rank_ref[...] = rankm
    rankt_ref[...] = rankm.T
    cnt_ref[...] = prefix.astype(jnp.int32)


def _moe_kernel(d_ff, cnt_ref, comb_ref, rank_ref, rankt_ref, xf_ref,
                wup_ref, bup_ref, wg_ref, bg_ref, wdn_ref, bdn_ref,
                out_ref, xblk_ref, acc_ref):
    e = pl.program_id(0)
    f = pl.program_id(1)
    nf = pl.num_programs(1)
    ce = cnt_ref[0, e]
    nb = (ce + _M - 1) // _M
    T = xf_ref.shape[0]

    @pl.when((e == 0) & (f == 0))
    def _init():
        out_ref[...] = jnp.zeros_like(out_ref)

    @pl.when(f == 0)
    def _gather():
        row = jax.lax.broadcasted_iota(jnp.int32, (1, T), 0)

        def erow(rt):
            return jnp.sum(
                jnp.where(jax.lax.broadcasted_iota(
                    jnp.int32, rt.shape, 0) == e, rt, 0.0),
                axis=0, keepdims=True)

        rank_row = erow(rankt_ref[...])
        riota = jax.lax.broadcasted_iota(
            jnp.int32, (_M, T), 0).astype(jnp.float32)

        def gath(rb, c):
            oh = (rank_row == riota + (rb * _M).astype(jnp.float32)
                  ).astype(jnp.float32)
            xg = jnp.dot(oh, xf_ref[...], preferred_element_type=jnp.float32)
            xblk_ref[pl.ds(rb * _M, _M), :] = xg
            return c

        jax.lax.fori_loop(0, nb, gath, 0)

    bup = bup_ref[0, 0]
    bg = bg_ref[0, 0]
    bdn = bdn_ref[0]
    isf0 = f == 0
    fcol = jax.lax.broadcasted_iota(jnp.int32, (1, _F), 1) + f * _F
    colvalid = fcol < d_ff
    frow = jax.lax.broadcasted_iota(jnp.int32, (_F, 1), 0) + f * _F
    wdn = jnp.where(frow < d_ff, wdn_ref[0], 0.0)

    def comp(rb, c):
        xb = xblk_ref[pl.ds(rb * _M, _M), :]
        u = jnp.dot(xb, wup_ref[0], preferred_element_type=jnp.float32) + bup
        g = jnp.dot(xb, wg_ref[0], preferred_element_type=jnp.float32) + bg
        h = u * (g * jax.lax.logistic(g))
        h = jnp.where(colvalid, h, 0.0)
        p = jnp.dot(h, wdn, preferred_element_type=jnp.float32)
        prev = acc_ref[pl.ds(rb * _M, _M), :]
        acc_ref[pl.ds(rb * _M, _M), :] = p + jnp.where(
            isf0, jnp.broadcast_to(bdn, p.shape), prev)
        return c

    jax.lax.fori_loop(0, nb, comp, 0)

    @pl.when(f == nf - 1)
    def _scatter():
        col16 = jax.lax.broadcasted_iota(jnp.int32, (T, _N_EXP), 1)
        rank_col = jnp.sum(
            jnp.where(col16 == e, rank_ref[...], 0.0), axis=1, keepdims=True)
        comb_col = jnp.sum(
            jnp.where(col16 == e, comb_ref[...], 0.0), axis=1, keepdims=True)
        siota = jax.lax.broadcasted_iota(
            jnp.int32, (T, _M), 1).astype(jnp.float32)

        def scat(rb, c):
            oh = (rank_col == siota + (rb * _M).astype(jnp.float32)
                  ).astype(jnp.float32)
            out_ref[...] += jnp.dot(oh * comb_col,
                                    acc_ref[pl.ds(rb * _M, _M), :],
                                    preferred_element_type=jnp.float32)
            return c

        jax.lax.fori_loop(0, nb, scat, 0)


def kernel(x, gate_W, gate_b, W_up, b_up, W_g, b_g, W_down, b_down):
    B, T, C = x.shape
    D_FF = W_up.shape[2]
    NF = (D_FF + _F - 1) // _F
    FP = NF * _F
    TP = ((T + _M - 1) // _M) * _M
    xf = x.reshape(T, C)
    bup_p = jnp.pad(b_up, ((0, 0), (0, FP - D_FF)))
    bg_p = jnp.pad(b_g, ((0, 0), (0, FP - D_FF)))

    comb, rankm, rankmt, cnt = pl.pallas_call(
        _router_kernel,
        out_shape=(jax.ShapeDtypeStruct((T, _N_EXP), jnp.float32),
                   jax.ShapeDtypeStruct((T, _N_EXP), jnp.float32),
                   jax.ShapeDtypeStruct((_N_EXP, T), jnp.float32),
                   jax.ShapeDtypeStruct((1, _N_EXP), jnp.int32)),
    )(xf, gate_W, gate_b.reshape(1, _N_EXP))

    smem = pl.BlockSpec(memory_space=pltpu.SMEM)
    out = pl.pallas_call(
        functools.partial(_moe_kernel, D_FF),
        grid=(_N_EXP, NF),
        in_specs=[
            smem,
            pl.BlockSpec((T, _N_EXP), lambda e, f: (0, 0)),
            pl.BlockSpec((T, _N_EXP), lambda e, f: (0, 0)),
            pl.BlockSpec((_N_EXP, T), lambda e, f: (0, 0)),
            pl.BlockSpec((T, C), lambda e, f: (0, 0)),
            pl.BlockSpec((1, C, _F), lambda e, f: (e, 0, f)),
            pl.BlockSpec((1, 1, 1, _F), lambda e, f: (e, f, 0, 0)),
            pl.BlockSpec((1, C, _F), lambda e, f: (e, 0, f)),
            pl.BlockSpec((1, 1, 1, _F), lambda e, f: (e, f, 0, 0)),
            pl.BlockSpec((1, _F, C), lambda e, f: (e, f, 0)),
            pl.BlockSpec((1, 1, C), lambda e, f: (e, 0, 0)),
        ],
        out_specs=pl.BlockSpec((T, C), lambda e, f: (0, 0)),
        out_shape=jax.ShapeDtypeStruct((T, C), jnp.float32),
        scratch_shapes=[pltpu.VMEM((TP, C), jnp.float32),
                        pltpu.VMEM((TP, C), jnp.float32)],
        compiler_params=pltpu.CompilerParams(
            vmem_limit_bytes=100 * 1024 * 1024),
    )(cnt, comb, rankm, rankmt, xf, W_up, bup_p.reshape(_N_EXP, NF, 1, _F),
      W_g, bg_p.reshape(_N_EXP, NF, 1, _F), W_down,
      b_down.reshape(_N_EXP, 1, C))

    return out.reshape(B, T, C)


# speedup vs baseline: 1.6220x; 1.0540x over previous
import functools

import jax
import jax.numpy as jnp
from jax.experimental import pallas as pl
from jax.experimental.pallas import tpu as pltpu

_N_EXP = 16
_TOP_K = 2
_M = 256
_F = 512


def _router_kernel(xf_ref, gw_ref, gb_ref, comb_ref, rank_ref, rankt_ref,
                   cnt_ref):
    scores = jnp.dot(xf_ref[...], gw_ref[...],
                     preferred_element_type=jnp.float32) + gb_ref[...]
    T = scores.shape[0]
    col = jax.lax.broadcasted_iota(jnp.int32, scores.shape, 1)
    m1 = jnp.max(scores, axis=1, keepdims=True)
    idx1 = jnp.min(jnp.where(scores == m1, col, _N_EXP), axis=1, keepdims=True)
    masked = jnp.where(col == idx1, -jnp.inf, scores)
    m2 = jnp.max(masked, axis=1, keepdims=True)
    idx2 = jnp.min(jnp.where(masked == m2, col, _N_EXP), axis=1, keepdims=True)
    oh1 = (col == idx1).astype(jnp.float32)
    oh2 = (col == idx2).astype(jnp.float32)
    mask = oh1 + oh2
    comb_ref[...] = oh1 * m1 + oh2 * m2

    CH = 256 if T % 256 == 0 else T
    r_i = jax.lax.broadcasted_iota(jnp.int32, (CH, CH), 0)
    c_i = jax.lax.broadcasted_iota(jnp.int32, (CH, CH), 1)
    ltri = (c_i < r_i).astype(jnp.float32)
    chunks = []
    prefix = jnp.zeros((1, _N_EXP), jnp.float32)
    for ci in range(T // CH):
        sub = mask[ci * CH:(ci + 1) * CH]
        cum = jnp.dot(ltri, sub, preferred_element_type=jnp.float32) + prefix
        chunks.append(cum)
        prefix = prefix + jnp.sum(sub, axis=0, keepdims=True)
    rank = jnp.concatenate(chunks, axis=0)
    rankm = jnp.where(mask > 0, rank, -1.0)
    rank_ref[...] = rankm
    rankt_ref[...] = rankm.T
    cnt_ref[...] = prefix.astype(jnp.int32)


def _moe_kernel(d_ff, cnt_ref, comb_ref, rank_ref, rankt_ref, xf_ref,
                wup_ref, bup_ref, wg_ref, bg_ref, wdn_ref, bdn_ref,
                out_ref, xblk_ref, acc_ref):
    e = pl.program_id(0)
    f = pl.program_id(1)
    nf = pl.num_programs(1)
    ce = cnt_ref[0, e]
    nb = (ce + _M - 1) // _M
    T = xf_ref.shape[0]

    @pl.when((e == 0) & (f == 0))
    def _init():
        out_ref[...] = jnp.zeros_like(out_ref)

    @pl.when(f == 0)
    def _gather():
        row = jax.lax.broadcasted_iota(jnp.int32, (1, T), 0)

        def erow(rt):
            return jnp.sum(
                jnp.where(jax.lax.broadcasted_iota(
                    jnp.int32, rt.shape, 0) == e, rt, 0.0),
                axis=0, keepdims=True)

        rank_row = erow(rankt_ref[...])
        riota = jax.lax.broadcasted_iota(
            jnp.int32, (_M, T), 0).astype(jnp.float32)

        def gath(rb, c):
            oh = (rank_row == riota + (rb * _M).astype(jnp.float32)
                  ).astype(jnp.float32)
            xg = jnp.dot(oh, xf_ref[...], preferred_element_type=jnp.float32)
            xblk_ref[pl.ds(rb * _M, _M), :] = xg
            return c

        jax.lax.fori_loop(0, nb, gath, 0)

    bup = bup_ref[0, 0]
    bg = bg_ref[0, 0]
    bdn = bdn_ref[0]
    isf0 = f == 0
    fcol = jax.lax.broadcasted_iota(jnp.int32, (1, _F), 1) + f * _F
    colvalid = fcol < d_ff
    frow = jax.lax.broadcasted_iota(jnp.int32, (_F, 1), 0) + f * _F
    wdn = jnp.where(frow < d_ff, wdn_ref[0], 0.0)

    def comp(rb, c):
        xb = xblk_ref[pl.ds(rb * _M, _M), :]
        u = jnp.dot(xb, wup_ref[0], preferred_element_type=jnp.float32) + bup
        g = jnp.dot(xb, wg_ref[0], preferred_element_type=jnp.float32) + bg
        h = u * (g * jax.lax.logistic(g))
        h = jnp.where(colvalid, h, 0.0)
        p = jnp.dot(h, wdn, preferred_element_type=jnp.float32)
        prev = acc_ref[pl.ds(rb * _M, _M), :]
        acc_ref[pl.ds(rb * _M, _M), :] = p + jnp.where(
            isf0, jnp.broadcast_to(bdn, p.shape), prev)
        return c

    jax.lax.fori_loop(0, nb, comp, 0)

    @pl.when(f == nf - 1)
    def _scatter():
        col16 = jax.lax.broadcasted_iota(jnp.int32, (T, _N_EXP), 1)
        rank_col = jnp.sum(
            jnp.where(col16 == e, rank_ref[...], 0.0), axis=1, keepdims=True)
        comb_col = jnp.sum(
            jnp.where(col16 == e, comb_ref[...], 0.0), axis=1, keepdims=True)
        siota = jax.lax.broadcasted_iota(
            jnp.int32, (T, _M), 1).astype(jnp.float32)

        def scat(rb, c):
            oh = (rank_col == siota + (rb * _M).astype(jnp.float32)
                  ).astype(jnp.float32)
            out_ref[...] += jnp.dot(oh * comb_col,
                                    acc_ref[pl.ds(rb * _M, _M), :],
                                    preferred_element_type=jnp.float32)
            return c

        jax.lax.fori_loop(0, nb, scat, 0)


def kernel(x, gate_W, gate_b, W_up, b_up, W_g, b_g, W_down, b_down):
    B, T, C = x.shape
    D_FF = W_up.shape[2]
    NF = (D_FF + _F - 1) // _F
    FP = NF * _F
    TP = ((T + _M - 1) // _M) * _M
    xf = x.reshape(T, C)
    bup_p = jnp.pad(b_up, ((0, 0), (0, FP - D_FF)))
    bg_p = jnp.pad(b_g, ((0, 0), (0, FP - D_FF)))

    comb, rankm, rankmt, cnt = pl.pallas_call(
        _router_kernel,
        out_shape=(jax.ShapeDtypeStruct((T, _N_EXP), jnp.float32),
                   jax.ShapeDtypeStruct((T, _N_EXP), jnp.float32),
                   jax.ShapeDtypeStruct((_N_EXP, T), jnp.float32),
                   jax.ShapeDtypeStruct((1, _N_EXP), jnp.int32)),
    )(xf, gate_W, gate_b.reshape(1, _N_EXP))

    smem = pl.BlockSpec(memory_space=pltpu.SMEM)
    out = pl.pallas_call(
        functools.partial(_moe_kernel, D_FF),
        grid=(_N_EXP, NF),
        in_specs=[
            smem,
            pl.BlockSpec((T, _N_EXP), lambda e, f: (0, 0)),
            pl.BlockSpec((T, _N_EXP), lambda e, f: (0, 0)),
            pl.BlockSpec((_N_EXP, T), lambda e, f: (0, 0)),
            pl.BlockSpec((T, C), lambda e, f: (0, 0)),
            pl.BlockSpec((1, C, _F), lambda e, f: (e, 0, f)),
            pl.BlockSpec((1, 1, 1, _F), lambda e, f: (e, f, 0, 0)),
            pl.BlockSpec((1, C, _F), lambda e, f: (e, 0, f)),
            pl.BlockSpec((1, 1, 1, _F), lambda e, f: (e, f, 0, 0)),
            pl.BlockSpec((1, _F, C), lambda e, f: (e, f, 0)),
            pl.BlockSpec((1, 1, C), lambda e, f: (e, 0, 0)),
        ],
        out_specs=pl.BlockSpec((T, C), lambda e, f: (0, 0)),
        out_shape=jax.ShapeDtypeStruct((T, C), jnp.float32),
        scratch_shapes=[pltpu.VMEM((TP, C), jnp.float32),
                        pltpu.VMEM((TP, C), jnp.float32)],
        compiler_params=pltpu.CompilerParams(
            vmem_limit_bytes=100 * 1024 * 1024),
    )(cnt, comb, rankm, rankmt, xf, W_up, bup_p.reshape(_N_EXP, NF, 1, _F),
      W_g, bg_p.reshape(_N_EXP, NF, 1, _F), W_down,
      b_down.reshape(_N_EXP, 1, C))

    return out.reshape(B, T, C)
